# baseline (speedup 1.0000x reference)
import functools

import jax
import jax.numpy as jnp
from jax import lax
from jax.experimental import pallas as pl
from jax.experimental.pallas import tpu as pltpu
from jax.experimental.pallas import tpu_sc as plsc

N = 500000
D = 5
P = 1000
P2 = 1024
SCALE = 100.0

NC = 2
NS = 16
NW = NC * NS

CHUNK = 400
NCHUNKS = N // CHUNK
GROUPS = CHUNK // 16
FULL = NCHUNKS // NW
EXTRA = NCHUNKS % NW

_mesh = plsc.VectorSubcoreMesh(core_axis_name="c", subcore_axis_name="s")


@functools.partial(
    pl.kernel,
    mesh=_mesh,
    compiler_params=pltpu.CompilerParams(needs_layout_passes=False),
    out_type=[
        jax.ShapeDtypeStruct((NW, P2), jnp.float32),
        jax.ShapeDtypeStruct((NW, P2), jnp.float32),
    ],
    scratch_types=[
        pltpu.VMEM((CHUNK, D), jnp.float32),
        pltpu.VMEM((CHUNK, D), jnp.float32),
        pltpu.VMEM((CHUNK,), jnp.int32),
        pltpu.VMEM((CHUNK,), jnp.int32),
        pltpu.VMEM((P2,), jnp.float32),
        pltpu.VMEM((P2,), jnp.float32),
    ],
)
def _sc_accum(pred_hbm, tp_hbm, pid_hbm, rec_hbm, sums_out, cnts_out,
              pbuf, tbuf, pidbuf, recbuf, sums, cnts):
    wid = lax.axis_index("c") * NS + lax.axis_index("s")

    zero16 = jnp.zeros((16,), jnp.float32)

    def _zero_body(i, _):
        sums[pl.ds(i * 16, 16)] = zero16
        cnts[pl.ds(i * 16, 16)] = zero16
        return 0

    lax.fori_loop(0, P2 // 16, _zero_body, 0)

    lanes = lax.iota(jnp.int32, 16)

    def _do_chunk(c):
        base = c * CHUNK
        pltpu.sync_copy(pred_hbm.at[pl.ds(base, CHUNK), :], pbuf)
        pltpu.sync_copy(tp_hbm.at[pl.ds(base, CHUNK), :], tbuf)
        pltpu.sync_copy(pid_hbm.at[pl.ds(base, CHUNK)], pidbuf)
        pltpu.sync_copy(rec_hbm.at[pl.ds(base, CHUNK)], recbuf)

        def _group_body(j, _):
            rows = j * 16 + lanes
            acc = jnp.zeros((16,), jnp.float32)
            for d in range(D):
                dcol = jnp.full((16,), d, jnp.int32)
                a = plsc.load_gather(pbuf, [rows, dcol])
                b = plsc.load_gather(tbuf, [rows, dcol])
                df = a - b
                acc = acc + df * df
            pid = pidbuf[pl.ds(j * 16, 16)]
            rec = recbuf[pl.ds(j * 16, 16)]
            valid = (rec > 0) & (pid > 0)
            pid_eff = jnp.where(valid, pid, 0)
            vf = valid.astype(jnp.float32)
            plsc.addupdate_scatter(sums, [pid_eff], acc * vf)
            plsc.addupdate_scatter(cnts, [pid_eff], vf)
            return 0

        lax.fori_loop(0, GROUPS, _group_body, 0)

    def _chunk_body(ci, _):
        _do_chunk(wid + ci * NW)
        return 0

    lax.fori_loop(0, FULL, _chunk_body, 0)

    @pl.when(wid < EXTRA)
    def _():
        _do_chunk(wid + FULL * NW)

    pltpu.sync_copy(sums, sums_out.at[wid])
    pltpu.sync_copy(cnts, cnts_out.at[wid])


def _finalize_body(sums_ref, cnts_ref, out_ref):
    s = jnp.sum(sums_ref[...], axis=0, keepdims=True)
    c = jnp.sum(cnts_ref[...], axis=0, keepdims=True)
    pid = lax.broadcasted_iota(jnp.int32, (1, P2), 1)
    present = (pid > 0) & (c > 0.0)
    denom = jnp.where(present, c, 1.0)
    terms = jnp.where(present, s / denom, 0.0)
    loss = jnp.sum(terms)
    k = jnp.sum(present.astype(jnp.float32))
    out_ref[...] = jnp.reshape(SCALE * loss / k, (1, 1))


def kernel(W, beta, H, pred, Y, particle_id, track_params, reconstructable):
    sums, cnts = _sc_accum(pred, track_params, particle_id, reconstructable)
    out = pl.pallas_call(
        _finalize_body,
        out_shape=jax.ShapeDtypeStruct((1, 1), jnp.float32),
    )(sums, cnts)
    return out[0, 0]


# speedup vs baseline: 1.1831x; 1.1831x over previous
import functools

import jax
import jax.numpy as jnp
from jax import lax
from jax.experimental import pallas as pl
from jax.experimental.pallas import tpu as pltpu
from jax.experimental.pallas import tpu_sc as plsc

N = 500000
D = 5
P = 1000
P2 = 1024
SCALE = 100.0

NC = 2
NS = 16
NW = NC * NS


HBLK = 8192
NBLK = (N + HBLK - 1) // HBLK


def _val_body(pred_ref, tp_ref, pid_ref, rec_ref, val_ref, pide_ref):
    d = pred_ref[...] - tp_ref[...]
    mse = jnp.sum(d * d, axis=1)
    pid = pid_ref[...]
    rec = rec_ref[...]
    valid = (rec > 0) & (pid > 0)
    val_ref[...] = jnp.where(valid, mse, 0.0)
    pide_ref[...] = jnp.where(valid, pid, 0)


_val_call = pl.pallas_call(
    _val_body,
    grid=(NBLK,),
    in_specs=[
        pl.BlockSpec((HBLK, D), lambda i: (i, 0)),
        pl.BlockSpec((HBLK, D), lambda i: (i, 0)),
        pl.BlockSpec((HBLK,), lambda i: (i,)),
        pl.BlockSpec((HBLK,), lambda i: (i,)),
    ],
    out_specs=[
        pl.BlockSpec((HBLK,), lambda i: (i,)),
        pl.BlockSpec((HBLK,), lambda i: (i,)),
    ],
    out_shape=[
        jax.ShapeDtypeStruct((N,), jnp.float32),
        jax.ShapeDtypeStruct((N,), jnp.int32),
    ],
)


CHUNK = 4000
NCHUNKS = N // CHUNK
GROUPS = CHUNK // 16
FULL = NCHUNKS // NW
EXTRA = NCHUNKS % NW

_mesh = plsc.VectorSubcoreMesh(core_axis_name="c", subcore_axis_name="s")


@functools.partial(
    pl.kernel,
    mesh=_mesh,
    compiler_params=pltpu.CompilerParams(needs_layout_passes=False),
    out_type=[
        jax.ShapeDtypeStruct((NW, P2), jnp.float32),
        jax.ShapeDtypeStruct((NW, P2), jnp.float32),
    ],
    scratch_types=[
        pltpu.VMEM((CHUNK,), jnp.float32),
        pltpu.VMEM((CHUNK,), jnp.int32),
        pltpu.VMEM((P2,), jnp.float32),
        pltpu.VMEM((P2,), jnp.float32),
    ],
)
def _sc_accum(val_hbm, pide_hbm, sums_out, cnts_out, vbuf, pbuf, sums, cnts):
    wid = lax.axis_index("c") * NS + lax.axis_index("s")

    zero16 = jnp.zeros((16,), jnp.float32)

    def _zero_body(i, _):
        sums[pl.ds(i * 16, 16)] = zero16
        cnts[pl.ds(i * 16, 16)] = zero16
        return 0

    lax.fori_loop(0, P2 // 16, _zero_body, 0)

    def _do_chunk(c):
        base = c * CHUNK
        pltpu.sync_copy(val_hbm.at[pl.ds(base, CHUNK)], vbuf)
        pltpu.sync_copy(pide_hbm.at[pl.ds(base, CHUNK)], pbuf)

        def _group_body(j, _):
            pid = pbuf[pl.ds(j * 16, 16)]
            v = vbuf[pl.ds(j * 16, 16)]
            vf = (pid > 0).astype(jnp.float32)
            plsc.addupdate_scatter(sums, [pid], v)
            plsc.addupdate_scatter(cnts, [pid], vf)
            return 0

        lax.fori_loop(0, GROUPS, _group_body, 0)

    def _chunk_body(ci, _):
        _do_chunk(wid + ci * NW)
        return 0

    lax.fori_loop(0, FULL, _chunk_body, 0)

    @pl.when(wid < EXTRA)
    def _():
        _do_chunk(wid + FULL * NW)

    pltpu.sync_copy(sums, sums_out.at[wid])
    pltpu.sync_copy(cnts, cnts_out.at[wid])


def _finalize_body(sums_ref, cnts_ref, out_ref):
    s = jnp.sum(sums_ref[...], axis=0, keepdims=True)
    c = jnp.sum(cnts_ref[...], axis=0, keepdims=True)
    pid = lax.broadcasted_iota(jnp.int32, (1, P2), 1)
    present = (pid > 0) & (c > 0.0)
    denom = jnp.where(present, c, 1.0)
    terms = jnp.where(present, s / denom, 0.0)
    loss = jnp.sum(terms)
    k = jnp.sum(present.astype(jnp.float32))
    out_ref[...] = jnp.reshape(SCALE * loss / k, (1, 1))


def kernel(W, beta, H, pred, Y, particle_id, track_params, reconstructable):
    val, pide = _val_call(pred, track_params, particle_id, reconstructable)
    sums, cnts = _sc_accum(val, pide)
    out = pl.pallas_call(
        _finalize_body,
        out_shape=jax.ShapeDtypeStruct((1, 1), jnp.float32),
    )(sums, cnts)
    return out[0, 0]


# speedup vs baseline: 7.3762x; 6.2345x over previous
import functools

import jax
import jax.numpy as jnp
from jax import lax
from jax.experimental import pallas as pl
from jax.experimental.pallas import tpu as pltpu
from jax.experimental.pallas import tpu_sc as plsc

N = 500000
D = 5
P = 1000
P2 = 1024
SCALE = 100.0

NC = 2
NS = 16
NW = NC * NS

CHUNK = 2048
NFULL = N // CHUNK
TAIL = N - NFULL * CHUNK
TAIL_BASE = NFULL * CHUNK
FULL = NFULL // NW
EXTRA = NFULL % NW
TAIL_WID = EXTRA

_mesh = plsc.VectorSubcoreMesh(core_axis_name="c", subcore_axis_name="s")


@functools.partial(
    pl.kernel,
    mesh=_mesh,
    compiler_params=pltpu.CompilerParams(needs_layout_passes=False),
    out_type=[
        jax.ShapeDtypeStruct((NW, P2), jnp.float32),
        jax.ShapeDtypeStruct((NW, P2), jnp.float32),
    ],
    scratch_types=[
        pltpu.VMEM((D, CHUNK), jnp.float32),
        pltpu.VMEM((D, CHUNK), jnp.float32),
        pltpu.VMEM((CHUNK,), jnp.int32),
        pltpu.VMEM((CHUNK,), jnp.int32),
        pltpu.VMEM((D, TAIL), jnp.float32),
        pltpu.VMEM((D, TAIL), jnp.float32),
        pltpu.VMEM((TAIL,), jnp.int32),
        pltpu.VMEM((TAIL,), jnp.int32),
        pltpu.VMEM((P2,), jnp.float32),
        pltpu.VMEM((P2,), jnp.float32),
    ],
)
def _sc_accum(predt_hbm, tpt_hbm, pid_hbm, rec_hbm, sums_out, cnts_out,
              pbuf, tbuf, pidbuf, recbuf, pbuf_t, tbuf_t, pidbuf_t, recbuf_t,
              sums, cnts):
    wid = lax.axis_index("c") * NS + lax.axis_index("s")

    zero16 = jnp.zeros((16,), jnp.float32)

    def _zero_body(i, _):
        sums[pl.ds(i * 16, 16)] = zero16
        cnts[pl.ds(i * 16, 16)] = zero16
        return 0

    lax.fori_loop(0, P2 // 16, _zero_body, 0)

    def _accum_groups(pb, tb, idb, rcb, groups):
        def _group_body(j, _):
            acc = jnp.zeros((16,), jnp.float32)
            for d in range(D):
                a = pb[d, pl.ds(j * 16, 16)]
                b = tb[d, pl.ds(j * 16, 16)]
                df = a - b
                acc = acc + df * df
            pid = idb[pl.ds(j * 16, 16)]
            rec = rcb[pl.ds(j * 16, 16)]
            valid = (rec > 0) & (pid > 0)
            pid_eff = jnp.where(valid, pid, 0)
            vf = valid.astype(jnp.float32)
            plsc.addupdate_scatter(sums, [pid_eff], acc * vf)
            plsc.addupdate_scatter(cnts, [pid_eff], vf)
            return 0

        lax.fori_loop(0, groups, _group_body, 0)

    def _do_chunk(c):
        base = c * CHUNK
        pltpu.sync_copy(predt_hbm.at[:, pl.ds(base, CHUNK)], pbuf)
        pltpu.sync_copy(tpt_hbm.at[:, pl.ds(base, CHUNK)], tbuf)
        pltpu.sync_copy(pid_hbm.at[pl.ds(base, CHUNK)], pidbuf)
        pltpu.sync_copy(rec_hbm.at[pl.ds(base, CHUNK)], recbuf)
        _accum_groups(pbuf, tbuf, pidbuf, recbuf, CHUNK // 16)

    def _chunk_body(ci, _):
        _do_chunk(wid + ci * NW)
        return 0

    lax.fori_loop(0, FULL, _chunk_body, 0)

    @pl.when(wid < EXTRA)
    def _():
        _do_chunk(wid + FULL * NW)

    @pl.when(wid == TAIL_WID)
    def _():
        pltpu.sync_copy(predt_hbm.at[:, pl.ds(TAIL_BASE, TAIL)], pbuf_t)
        pltpu.sync_copy(tpt_hbm.at[:, pl.ds(TAIL_BASE, TAIL)], tbuf_t)
        pltpu.sync_copy(pid_hbm.at[pl.ds(TAIL_BASE, TAIL)], pidbuf_t)
        pltpu.sync_copy(rec_hbm.at[pl.ds(TAIL_BASE, TAIL)], recbuf_t)
        _accum_groups(pbuf_t, tbuf_t, pidbuf_t, recbuf_t, TAIL // 16)

    pltpu.sync_copy(sums, sums_out.at[wid])
    pltpu.sync_copy(cnts, cnts_out.at[wid])


def _finalize_body(sums_ref, cnts_ref, out_ref):
    s = jnp.sum(sums_ref[...], axis=0, keepdims=True)
    c = jnp.sum(cnts_ref[...], axis=0, keepdims=True)
    pid = lax.broadcasted_iota(jnp.int32, (1, P2), 1)
    present = (pid > 0) & (c > 0.0)
    denom = jnp.where(present, c, 1.0)
    terms = jnp.where(present, s / denom, 0.0)
    loss = jnp.sum(terms)
    k = jnp.sum(present.astype(jnp.float32))
    out_ref[...] = jnp.reshape(SCALE * loss / k, (1, 1))


def kernel(W, beta, H, pred, Y, particle_id, track_params, reconstructable):
    predt = jnp.transpose(pred)
    tpt = jnp.transpose(track_params)
    sums, cnts = _sc_accum(predt, tpt, particle_id, reconstructable)
    out = pl.pallas_call(
        _finalize_body,
        out_shape=jax.ShapeDtypeStruct((1, 1), jnp.float32),
    )(sums, cnts)
    return out[0, 0]


# speedup vs baseline: 10.5093x; 1.4248x over previous
import functools

import jax
import jax.numpy as jnp
from jax import lax
from jax.experimental import pallas as pl
from jax.experimental.pallas import tpu as pltpu
from jax.experimental.pallas import tpu_sc as plsc

N = 500000
D = 5
P = 1000
P2 = 1024
SCALE = 100.0

NC = 2
NS = 16
NW = NC * NS

CHUNK = 2048
NFULL = N // CHUNK
TAIL = N - NFULL * CHUNK
TAIL_BASE = NFULL * CHUNK
FULL = NFULL // NW
EXTRA = NFULL % NW
TAIL_WID = EXTRA

_mesh = plsc.VectorSubcoreMesh(core_axis_name="c", subcore_axis_name="s")

_CHUNK_BUFS = [
    pltpu.VMEM((D, CHUNK), jnp.float32),
    pltpu.VMEM((D, CHUNK), jnp.float32),
    pltpu.VMEM((CHUNK,), jnp.int32),
    pltpu.VMEM((CHUNK,), jnp.int32),
]


@functools.partial(
    pl.kernel,
    mesh=_mesh,
    compiler_params=pltpu.CompilerParams(needs_layout_passes=False),
    out_type=[
        jax.ShapeDtypeStruct((NW, P2), jnp.float32),
        jax.ShapeDtypeStruct((NW, P2), jnp.float32),
    ],
    scratch_types=(
        _CHUNK_BUFS
        + _CHUNK_BUFS
        + _CHUNK_BUFS
        + [
            pltpu.VMEM((D, TAIL), jnp.float32),
            pltpu.VMEM((D, TAIL), jnp.float32),
            pltpu.VMEM((TAIL,), jnp.int32),
            pltpu.VMEM((TAIL,), jnp.int32),
            pltpu.VMEM((P2,), jnp.float32),
            pltpu.VMEM((P2,), jnp.float32),
            pltpu.SemaphoreType.DMA,
            pltpu.SemaphoreType.DMA,
            pltpu.SemaphoreType.DMA,
        ]
    ),
)
def _sc_accum(predt_hbm, tpt_hbm, pid_hbm, rec_hbm, sums_out, cnts_out,
              pbuf0, tbuf0, pidbuf0, recbuf0,
              pbuf1, tbuf1, pidbuf1, recbuf1,
              pbuf2, tbuf2, pidbuf2, recbuf2,
              pbuf_t, tbuf_t, pidbuf_t, recbuf_t,
              sums, cnts, sem0, sem1, sem2):
    wid = lax.axis_index("c") * NS + lax.axis_index("s")

    sets = [
        (pbuf0, tbuf0, pidbuf0, recbuf0, sem0),
        (pbuf1, tbuf1, pidbuf1, recbuf1, sem1),
    ]

    def _start(c, pb, tb, idb, rcb, sem):
        base = c * CHUNK
        return [
            pltpu.async_copy(predt_hbm.at[:, pl.ds(base, CHUNK)], pb, sem),
            pltpu.async_copy(tpt_hbm.at[:, pl.ds(base, CHUNK)], tb, sem),
            pltpu.async_copy(pid_hbm.at[pl.ds(base, CHUNK)], idb, sem),
            pltpu.async_copy(rec_hbm.at[pl.ds(base, CHUNK)], rcb, sem),
        ]

    @pl.when(wid < EXTRA)
    def _():
        _start(wid + FULL * NW, pbuf2, tbuf2, pidbuf2, recbuf2, sem2)

    handles = [None, None]
    handles[0] = _start(wid, *sets[0])

    zero16 = jnp.zeros((16,), jnp.float32)

    def _zero_body(i, _):
        sums[pl.ds(i * 16, 16)] = zero16
        cnts[pl.ds(i * 16, 16)] = zero16
        return 0

    lax.fori_loop(0, P2 // 16, _zero_body, 0)

    def _accum_groups(pb, tb, idb, rcb, groups):
        def _group_body(j, _):
            acc = jnp.zeros((16,), jnp.float32)
            for d in range(D):
                a = pb[d, pl.ds(j * 16, 16)]
                b = tb[d, pl.ds(j * 16, 16)]
                df = a - b
                acc = acc + df * df
            pid = idb[pl.ds(j * 16, 16)]
            rec = rcb[pl.ds(j * 16, 16)]
            valid = (rec > 0) & (pid > 0)
            pid_eff = jnp.where(valid, pid, 0)
            vf = valid.astype(jnp.float32)
            plsc.addupdate_scatter(sums, [pid_eff], acc * vf)
            plsc.addupdate_scatter(cnts, [pid_eff], vf)
            return 0

        lax.fori_loop(0, groups, _group_body, 0)

    for ci in range(FULL):
        b = ci % 2
        if ci + 1 < FULL:
            handles[1 - b] = _start(wid + (ci + 1) * NW, *sets[1 - b])
        for h in handles[b]:
            h.wait()
        pb, tb, idb, rcb, _ = sets[b]
        _accum_groups(pb, tb, idb, rcb, CHUNK // 16)

    @pl.when(wid < EXTRA)
    def _():
        pltpu.make_async_copy(
            predt_hbm.at[:, pl.ds(0, CHUNK)], pbuf2, sem2).wait()
        pltpu.make_async_copy(
            tpt_hbm.at[:, pl.ds(0, CHUNK)], tbuf2, sem2).wait()
        pltpu.make_async_copy(pid_hbm.at[pl.ds(0, CHUNK)], pidbuf2, sem2).wait()
        pltpu.make_async_copy(rec_hbm.at[pl.ds(0, CHUNK)], recbuf2, sem2).wait()
        _accum_groups(pbuf2, tbuf2, pidbuf2, recbuf2, CHUNK // 16)

    @pl.when(wid == TAIL_WID)
    def _():
        pltpu.sync_copy(predt_hbm.at[:, pl.ds(TAIL_BASE, TAIL)], pbuf_t)
        pltpu.sync_copy(tpt_hbm.at[:, pl.ds(TAIL_BASE, TAIL)], tbuf_t)
        pltpu.sync_copy(pid_hbm.at[pl.ds(TAIL_BASE, TAIL)], pidbuf_t)
        pltpu.sync_copy(rec_hbm.at[pl.ds(TAIL_BASE, TAIL)], recbuf_t)
        _accum_groups(pbuf_t, tbuf_t, pidbuf_t, recbuf_t, TAIL // 16)

    pltpu.sync_copy(sums, sums_out.at[wid])
    pltpu.sync_copy(cnts, cnts_out.at[wid])


def _finalize_body(sums_ref, cnts_ref, out_ref):
    s = jnp.sum(sums_ref[...], axis=0, keepdims=True)
    c = jnp.sum(cnts_ref[...], axis=0, keepdims=True)
    pid = lax.broadcasted_iota(jnp.int32, (1, P2), 1)
    present = (pid > 0) & (c > 0.0)
    denom = jnp.where(present, c, 1.0)
    terms = jnp.where(present, s / denom, 0.0)
    loss = jnp.sum(terms)
    k = jnp.sum(present.astype(jnp.float32))
    out_ref[...] = jnp.reshape(SCALE * loss / k, (1, 1))


def kernel(W, beta, H, pred, Y, particle_id, track_params, reconstructable):
    predt = jnp.transpose(pred)
    tpt = jnp.transpose(track_params)
    sums, cnts = _sc_accum(predt, tpt, particle_id, reconstructable)
    out = pl.pallas_call(
        _finalize_body,
        out_shape=jax.ShapeDtypeStruct((1, 1), jnp.float32),
    )(sums, cnts)
    return out[0, 0]
